# baseline (speedup 1.0000x reference)
import functools

import jax
import jax.numpy as jnp
from jax.experimental import pallas as pl
from jax.experimental.pallas import tpu as pltpu

TOPK = 5
MIN_SIM = 0.2
NEG = -3.0
QB = 256
XB = 1024
NCH = XB // 128
QSUB = 8
IDB = 9


def _sim_kernel(q_ref, x_ref, kb_ref, ss_ref, res_ref, s_ref, rs_ref, rk_ref,
                *, nx, nq, n_valid):
    xi = pl.program_id(0)
    qi = pl.program_id(1)

    @pl.when(xi == 0)
    def _init():
        rs_ref[qi] = jnp.full((TOPK, QB, 128), NEG, jnp.float32)
        rk_ref[qi] = jnp.zeros((TOPK, QB, 128), jnp.int32)

    s = jax.lax.dot_general(q_ref[...], x_ref[...], (((1,), (1,)), ((), ())),
                            preferred_element_type=jnp.float32,
                            precision=jax.lax.Precision.DEFAULT)

    @pl.when(xi < nx - 1)
    def _store_full():
        s_ref[...] = s

    @pl.when(xi == nx - 1)
    def _store_masked():
        gcol = jax.lax.iota(jnp.int32, XB) + xi * XB
        s_ref[...] = jnp.where((gcol < n_valid)[None, :], s, NEG)

    def body(qs, carry):
        b = qs * QSUB
        sv = [rs_ref[qi, j, pl.ds(b, QSUB), :] for j in range(TOPK)]
        sk = [rk_ref[qi, j, pl.ds(b, QSUB), :] for j in range(TOPK)]
        for c in range(NCH):
            v = s_ref[pl.ds(b, QSUB), c * 128:(c + 1) * 128]
            kv = kb_ref[0, :, c * 128:(c + 1) * 128]
            for j in range(TOPK):
                gt = v > sv[j]
                nv = jnp.minimum(v, sv[j])
                nk = jnp.where(gt, sk[j], kv)
                sv[j] = jnp.maximum(v, sv[j])
                sk[j] = jnp.where(gt, kv, sk[j])
                v, kv = nv, nk
        for j in range(TOPK):
            rs_ref[qi, j, pl.ds(b, QSUB), :] = sv[j]
            rk_ref[qi, j, pl.ds(b, QSUB), :] = sk[j]
        return carry

    jax.lax.fori_loop(0, QB // QSUB, body, 0)

    @pl.when(xi == nx - 1)
    def _epilogue():
        cat_s = jnp.concatenate([rs_ref[qi, j] for j in range(TOPK)], axis=1)
        cat_k = jnp.concatenate([rk_ref[qi, j] for j in range(TOPK)], axis=1)
        big = jnp.int32((nx * XB) << IDB)
        sims, keys = [], []
        for _ in range(TOPK):
            m = jnp.max(cat_s, axis=1, keepdims=True)
            key = jnp.min(jnp.where(cat_s == m, cat_k, big),
                          axis=1, keepdims=True)
            cat_s = jnp.where(cat_k == key, NEG, cat_s)
            sims.append(m)
            keys.append(key)
        ids = [(k & ((1 << IDB) - 1)).astype(jnp.float32) for k in keys]
        mask = [sv >= MIN_SIM for sv in sims]
        zero = jnp.zeros((QB, 1), jnp.float32)
        counts = []
        for i in range(TOPK):
            c = zero
            for j in range(TOPK):
                c = c + jnp.where(mask[j] & (ids[i] == ids[j]), 1.0, 0.0)
            counts.append(c)
        maxc = zero
        for i in range(TOPK):
            maxc = jnp.maximum(maxc, jnp.where(mask[i], counts[i], 0.0))
        bid = jnp.full((QB, 1), 1e9, jnp.float32)
        for i in range(TOPK):
            sel = mask[i] & (counts[i] == maxc)
            bid = jnp.minimum(bid, jnp.where(sel, ids[i], 1e9))
        resf = jnp.where(maxc > 0, bid, -1.0)
        ss = zero
        for i in range(TOPK):
            sel = mask[i] & (ids[i] == resf)
            ss = jnp.maximum(ss, jnp.where(sel, sims[i], 0.0))
        ss_ref[...] = ss[:, 0]
        res_ref[...] = resf[:, 0].astype(jnp.int32)


@jax.jit
def _run(descriptors, xmat, key_blocks):
    nq = descriptors.shape[0] // QB
    nx = key_blocks.shape[0]
    n_valid = xmat.shape[0]
    grid = (nx, nq)
    kfn = functools.partial(_sim_kernel, nx=nx, nq=nq, n_valid=n_valid)
    ss, res = pl.pallas_call(
        kfn,
        grid=grid,
        in_specs=[
            pl.BlockSpec((QB, 128), lambda xi, qi: (qi, 0)),
            pl.BlockSpec((XB, 128), lambda xi, qi: (xi, 0)),
            pl.BlockSpec((1, QSUB, XB), lambda xi, qi: (xi, 0, 0)),
        ],
        out_specs=[
            pl.BlockSpec((QB,), lambda xi, qi: (qi,)),
            pl.BlockSpec((QB,), lambda xi, qi: (qi,)),
        ],
        out_shape=[
            jax.ShapeDtypeStruct((descriptors.shape[0],), jnp.float32),
            jax.ShapeDtypeStruct((descriptors.shape[0],), jnp.int32),
        ],
        scratch_shapes=[
            pltpu.VMEM((QB, XB), jnp.float32),
            pltpu.VMEM((4, TOPK, QB, 128), jnp.float32),
            pltpu.VMEM((4, TOPK, QB, 128), jnp.int32),
        ],
    )(descriptors, xmat, key_blocks)
    return ss, res


def kernel(final_boxes, descriptors, places_db):
    xmat = places_db[:, :-1]
    ids = places_db[:, -1].astype(jnp.int32)
    n = xmat.shape[0]
    nx = pl.cdiv(n, XB)
    keys = (jax.lax.iota(jnp.int32, n) << IDB) | ids
    keys = jnp.pad(keys, (0, nx * XB - n)).reshape(nx, 1, XB)
    key_blocks = jnp.broadcast_to(keys, (nx, QSUB, XB))
    ss, res = _run(descriptors, xmat, key_blocks)
    return (final_boxes, ss, res)


# speedup vs baseline: 1.1031x; 1.1031x over previous
import functools

import jax
import jax.numpy as jnp
from jax.experimental import pallas as pl
from jax.experimental.pallas import tpu as pltpu

TOPK = 5
MIN_SIM = 0.2
NEG = -3.0
QB = 256
XB = 2048
NCH = XB // 128
QSUB = 8
IDB = 9


def _sim_kernel(q_ref, x_ref, kb_ref, ss_ref, res_ref, s_ref, rs_ref, rk_ref,
                *, nx, nq, n_valid):
    xi = pl.program_id(0)
    qi = pl.program_id(1)

    @pl.when(xi == 0)
    def _init():
        rs_ref[qi] = jnp.full((TOPK, QB, 128), NEG, jnp.float32)
        rk_ref[qi] = jnp.zeros((TOPK, QB, 128), jnp.int32)

    s = jax.lax.dot_general(q_ref[...], x_ref[...], (((1,), (1,)), ((), ())),
                            preferred_element_type=jnp.float32,
                            precision=jax.lax.Precision.DEFAULT)

    @pl.when(xi < nx - 1)
    def _store_full():
        s_ref[...] = s

    @pl.when(xi == nx - 1)
    def _store_masked():
        gcol = jax.lax.iota(jnp.int32, XB) + xi * XB
        s_ref[...] = jnp.where((gcol < n_valid)[None, :], s, NEG)

    def body(qs, carry):
        b = qs * QSUB
        sv = [rs_ref[qi, j, pl.ds(b, QSUB), :] for j in range(TOPK)]
        sk = [rk_ref[qi, j, pl.ds(b, QSUB), :] for j in range(TOPK)]
        for c in range(NCH):
            v = s_ref[pl.ds(b, QSUB), c * 128:(c + 1) * 128]
            kv = kb_ref[0, :, c * 128:(c + 1) * 128]
            for j in range(TOPK):
                gt = v > sv[j]
                nv = jnp.minimum(v, sv[j])
                nk = jnp.where(gt, sk[j], kv)
                sv[j] = jnp.maximum(v, sv[j])
                sk[j] = jnp.where(gt, kv, sk[j])
                v, kv = nv, nk
        for j in range(TOPK):
            rs_ref[qi, j, pl.ds(b, QSUB), :] = sv[j]
            rk_ref[qi, j, pl.ds(b, QSUB), :] = sk[j]
        return carry

    jax.lax.fori_loop(0, QB // QSUB, body, 0)

    @pl.when(xi == nx - 1)
    def _epilogue():
        cat_s = jnp.concatenate([rs_ref[qi, j] for j in range(TOPK)], axis=1)
        cat_k = jnp.concatenate([rk_ref[qi, j] for j in range(TOPK)], axis=1)
        big = jnp.int32((nx * XB) << IDB)
        sims, keys = [], []
        for _ in range(TOPK):
            m = jnp.max(cat_s, axis=1, keepdims=True)
            key = jnp.min(jnp.where(cat_s == m, cat_k, big),
                          axis=1, keepdims=True)
            cat_s = jnp.where(cat_k == key, NEG, cat_s)
            sims.append(m)
            keys.append(key)
        ids = [(k & ((1 << IDB) - 1)).astype(jnp.float32) for k in keys]
        mask = [sv >= MIN_SIM for sv in sims]
        zero = jnp.zeros((QB, 1), jnp.float32)
        counts = []
        for i in range(TOPK):
            c = zero
            for j in range(TOPK):
                c = c + jnp.where(mask[j] & (ids[i] == ids[j]), 1.0, 0.0)
            counts.append(c)
        maxc = zero
        for i in range(TOPK):
            maxc = jnp.maximum(maxc, jnp.where(mask[i], counts[i], 0.0))
        bid = jnp.full((QB, 1), 1e9, jnp.float32)
        for i in range(TOPK):
            sel = mask[i] & (counts[i] == maxc)
            bid = jnp.minimum(bid, jnp.where(sel, ids[i], 1e9))
        resf = jnp.where(maxc > 0, bid, -1.0)
        ss = zero
        for i in range(TOPK):
            sel = mask[i] & (ids[i] == resf)
            ss = jnp.maximum(ss, jnp.where(sel, sims[i], 0.0))
        ss_ref[...] = ss[:, 0]
        res_ref[...] = resf[:, 0].astype(jnp.int32)


@jax.jit
def _run(descriptors, xmat, key_blocks):
    nq = descriptors.shape[0] // QB
    nx = key_blocks.shape[0]
    n_valid = xmat.shape[0]
    grid = (nx, nq)
    kfn = functools.partial(_sim_kernel, nx=nx, nq=nq, n_valid=n_valid)
    ss, res = pl.pallas_call(
        kfn,
        grid=grid,
        in_specs=[
            pl.BlockSpec((QB, 128), lambda xi, qi: (qi, 0)),
            pl.BlockSpec((XB, 128), lambda xi, qi: (xi, 0)),
            pl.BlockSpec((1, QSUB, XB), lambda xi, qi: (xi, 0, 0)),
        ],
        out_specs=[
            pl.BlockSpec((QB,), lambda xi, qi: (qi,)),
            pl.BlockSpec((QB,), lambda xi, qi: (qi,)),
        ],
        out_shape=[
            jax.ShapeDtypeStruct((descriptors.shape[0],), jnp.float32),
            jax.ShapeDtypeStruct((descriptors.shape[0],), jnp.int32),
        ],
        scratch_shapes=[
            pltpu.VMEM((QB, XB), jnp.float32),
            pltpu.VMEM((4, TOPK, QB, 128), jnp.float32),
            pltpu.VMEM((4, TOPK, QB, 128), jnp.int32),
        ],
    )(descriptors, xmat, key_blocks)
    return ss, res


def kernel(final_boxes, descriptors, places_db):
    xmat = places_db[:, :-1]
    ids = places_db[:, -1].astype(jnp.int32)
    n = xmat.shape[0]
    nx = pl.cdiv(n, XB)
    keys = (jax.lax.iota(jnp.int32, n) << IDB) | ids
    keys = jnp.pad(keys, (0, nx * XB - n)).reshape(nx, 1, XB)
    key_blocks = jnp.broadcast_to(keys, (nx, QSUB, XB))
    ss, res = _run(descriptors, xmat, key_blocks)
    return (final_boxes, ss, res)
